# Initial kernel scaffold; baseline (speedup 1.0000x reference)
import jax
import jax.numpy as jnp
from jax.experimental import pallas as pl


def kernel(x, pos, edge_index, W_stem, b_stem, g_stem, be_stem, W1, b1, g1, be1, W2, b2, g2, be2, W_reg, b_reg, W_obj, b_obj, W_cls, b_cls):
    raise NotImplementedError("write your pallas kernel here")


# speedup vs baseline: 2.2010x; 2.2010x over previous
import functools

import jax
import jax.numpy as jnp
from jax import lax
from jax.experimental import pallas as pl
from jax.experimental.pallas import tpu as pltpu
from jax.experimental.pallas import tpu_sc as plsc

N = 10000
E = 320000
C = 128
EPS = 1e-5

NC = 2
NS = 16
NW = NC * NS
KCH = 4
CHUNK = 8000
NVEC = CHUNK // 16
NCHUNK = E // CHUNK

_NEG_INF = float("-inf")


@functools.lru_cache(maxsize=None)
def _make_segmax(G):
    rows = G * 128
    mesh = plsc.VectorSubcoreMesh(core_axis_name="c", subcore_axis_name="s",
                                  num_cores=NC, num_subcores=NS)

    @functools.partial(
        pl.kernel,
        out_type=jax.ShapeDtypeStruct((rows, N), jnp.float32),
        mesh=mesh,
        compiler_params=pltpu.CompilerParams(needs_layout_passes=False),
        scratch_types=(
            [pltpu.VMEM((N,), jnp.float32) for _ in range(KCH)]
            + [pltpu.VMEM((N,), jnp.float32) for _ in range(KCH)]
            + [
                pltpu.VMEM((CHUNK,), jnp.int32),
                pltpu.VMEM((CHUNK,), jnp.int32),
                pltpu.VMEM((CHUNK,), jnp.int32),
                pltpu.VMEM((CHUNK,), jnp.int32),
                pltpu.SemaphoreType.DMA,
                pltpu.SemaphoreType.DMA,
                pltpu.SemaphoreType.DMA,
                pltpu.SemaphoreType.DMA,
            ]
        ),
    )
    def seg(a_hbm, src_hbm, dst_hbm, out_hbm,
            a0, a1, a2, a3, q0, q1, q2, q3,
            src0_v, dst0_v, src1_v, dst1_v, ss0, sd0, ss1, sd1):
        a_refs = (a0, a1, a2, a3)
        acc_refs = (q0, q1, q2, q3)
        wid = lax.axis_index("s") * NC + lax.axis_index("c")
        ninf = jnp.full((16,), _NEG_INF, jnp.float32)
        bufs = ((src0_v, dst0_v, ss0, sd0), (src1_v, dst1_v, ss1, sd1))

        def issue(ci, b):
            sbuf, dbuf, ssem, dsem = bufs[b]
            pltpu.async_copy(src_hbm.at[pl.ds(ci * CHUNK, CHUNK)], sbuf, ssem)
            pltpu.async_copy(dst_hbm.at[pl.ds(ci * CHUNK, CHUNK)], dbuf, dsem)

        def drain(b):
            sbuf, dbuf, ssem, dsem = bufs[b]
            pltpu.make_async_copy(src_hbm.at[pl.ds(0, CHUNK)], sbuf,
                                  ssem).wait()
            pltpu.make_async_copy(dst_hbm.at[pl.ds(0, CHUNK)], dbuf,
                                  dsem).wait()

        issue(0, 0)
        issue(1, 1)

        def do_vec(sbuf, dbuf, vi):
            s16 = sbuf[pl.ds(vi * 16, 16)]
            d16 = dbuf[pl.ds(vi * 16, 16)]
            _, last = plsc.scan_count(d16)
            vals = []
            for c in range(KCH):
                val = plsc.load_gather(a_refs[c], [s16])
                cur = plsc.load_gather(acc_refs[c], [d16])
                plsc.store_scatter(acc_refs[c], [d16], val, mask=val > cur)
                vals.append(val)
            return d16, vals, ~last

        def fix_vec(d16, vals):
            for c in range(KCH):
                val = vals[c]

                def body(_, c=c, val=val):
                    cur = plsc.load_gather(acc_refs[c], [d16])
                    plsc.store_scatter(acc_refs[c], [d16], val,
                                       mask=val > cur)
                    cur2 = plsc.load_gather(acc_refs[c], [d16])
                    return jnp.any(val > cur2)

                lax.while_loop(lambda p: p, body, jnp.bool_(True))

        for g in range(G):
            row0 = g * 128 + wid * KCH
            for c in range(KCH):
                pltpu.sync_copy(a_hbm.at[row0 + c], a_refs[c])

                @pl.loop(0, N // 16, unroll=8)
                def _init(i, c=c):
                    acc_refs[c][pl.ds(i * 16, 16)] = ninf

            @pl.loop(0, NCHUNK // 2)
            def _pair(ci2):
                for b in range(2):
                    ci = ci2 * 2 + b
                    drain(b)
                    sbuf, dbuf = bufs[b][0], bufs[b][1]

                    @pl.loop(0, NVEC // 4)
                    def _vec(vj, sbuf=sbuf, dbuf=dbuf):
                        res = [do_vec(sbuf, dbuf, vj * 4 + u)
                               for u in range(4)]
                        dup = jnp.any(res[0][2] | res[1][2]
                                      | res[2][2] | res[3][2])

                        @pl.when(dup)
                        def _slow():
                            for d16, vals, _ in res:
                                fix_vec(d16, vals)

                    issue(lax.rem(ci + 2, NCHUNK), b)

            for c in range(KCH):
                pltpu.sync_copy(acc_refs[c], out_hbm.at[row0 + c])

        drain(0)
        drain(1)

    return seg


def _segmax1(a_t, src, dst):
    return _make_segmax(1)(a_t, src, dst)


def _segmax2(a_t, src, dst):
    return _make_segmax(2)(a_t, src, dst)


def _bn_relu(post, g, be):
    mean = jnp.mean(post, axis=0, keepdims=True)
    var = jnp.mean((post - mean) ** 2, axis=0, keepdims=True)
    return jnp.maximum((post - mean) / jnp.sqrt(var + EPS) * g + be, 0.0)


def _finite(s, d):
    o = s + d
    return jnp.where(jnp.isfinite(o), o, 0.0)


def _dense1_body(x_ref, p_ref, wx_ref, wp_ref, b_ref, a_ref, d_ref):
    pb = jnp.dot(p_ref[...], wp_ref[...], preferred_element_type=jnp.float32)
    a_ref[...] = jnp.dot(x_ref[...], wx_ref[...],
                         preferred_element_type=jnp.float32) + pb
    d_ref[...] = b_ref[...] - pb


@jax.jit
def _dense1(x, pos2, wx, wp, b):
    return pl.pallas_call(
        _dense1_body,
        out_shape=(jax.ShapeDtypeStruct((N, C), jnp.float32),
                   jax.ShapeDtypeStruct((N, C), jnp.float32)),
    )(x, pos2, wx, wp, b)


def _dense2_body(s_ref, dt_ref, g_ref, be_ref, p_ref,
                 wx1_ref, wp1_ref, b1_ref, wx2_ref, wp2_ref, b2_ref,
                 a1_ref, a2_ref, d1_ref, d2_ref):
    h = _bn_relu(_finite(s_ref[...], dt_ref[...]), g_ref[...], be_ref[...])
    pb1 = jnp.dot(p_ref[...], wp1_ref[...], preferred_element_type=jnp.float32)
    a1_ref[...] = jnp.dot(h, wx1_ref[...],
                          preferred_element_type=jnp.float32) + pb1
    d1_ref[...] = b1_ref[...] - pb1
    pb2 = jnp.dot(p_ref[...], wp2_ref[...], preferred_element_type=jnp.float32)
    a2_ref[...] = jnp.dot(h, wx2_ref[...],
                          preferred_element_type=jnp.float32) + pb2
    d2_ref[...] = b2_ref[...] - pb2


@jax.jit
def _dense2(s, dterm, g, be, pos2, wx1, wp1, b1, wx2, wp2, b2):
    return pl.pallas_call(
        _dense2_body,
        out_shape=tuple(jax.ShapeDtypeStruct((N, C), jnp.float32)
                        for _ in range(4)),
    )(s, dterm, g, be, pos2, wx1, wp1, b1, wx2, wp2, b2)


def _dense3_body(s1_ref, d1_ref, g1_ref, be1_ref, s2_ref, d2_ref, g2_ref,
                 be2_ref, p_ref, p1_ref, p2_ref, pp_ref, bh_ref,
                 ah_ref, dh_ref):
    x1 = _bn_relu(_finite(s1_ref[...], d1_ref[...]), g1_ref[...], be1_ref[...])
    x2 = _bn_relu(_finite(s2_ref[...], d2_ref[...]), g2_ref[...], be2_ref[...])
    pb = jnp.dot(p_ref[...], pp_ref[...], preferred_element_type=jnp.float32)
    ah_ref[...] = (jnp.dot(x1, p1_ref[...], preferred_element_type=jnp.float32)
                   + jnp.dot(x2, p2_ref[...],
                             preferred_element_type=jnp.float32) + pb)
    dh_ref[...] = bh_ref[...] - pb


@jax.jit
def _dense3(s1, d1, g1, be1, s2, d2, g2, be2, pos2, p1, p2, pp, bh):
    return pl.pallas_call(
        _dense3_body,
        out_shape=(jax.ShapeDtypeStruct((N, C), jnp.float32),
                   jax.ShapeDtypeStruct((N, C), jnp.float32)),
    )(s1, d1, g1, be1, s2, d2, g2, be2, pos2, p1, p2, pp, bh)


def _final_body(s_ref, d_ref, o_ref):
    o_ref[...] = _finite(s_ref[...], d_ref[...])


@jax.jit
def _final(s, d):
    return pl.pallas_call(
        _final_body,
        out_shape=jax.ShapeDtypeStruct((N, C), jnp.float32),
    )(s, d)


def kernel(x, pos, edge_index, W_stem, b_stem, g_stem, be_stem, W1, b1, g1,
           be1, W2, b2, g2, be2, W_reg, b_reg, W_obj, b_obj, W_cls, b_cls):
    pos2 = pos[:, :2]
    src = edge_index[0]
    dst = edge_index[1]

    a_s, d_s = _dense1(x, pos2, W_stem[:C], W_stem[C:], b_stem.reshape(1, C))
    s_st = _segmax1(a_s.T, src, dst)

    a1, a2, d1, d2 = _dense2(s_st.T, d_s, g_stem.reshape(1, C),
                             be_stem.reshape(1, C), pos2,
                             W1[:C], W1[C:], b1.reshape(1, C),
                             W2[:C], W2[C:], b2.reshape(1, C))
    a_mid_t = jnp.concatenate([a1.T, a2.T], axis=0)
    s_mt = _segmax2(a_mid_t, src, dst)

    p1 = jnp.zeros((C, C), jnp.float32)
    p1 = p1.at[:, 0:4].set(W_reg[:C]).at[:, 4:5].set(W_obj[:C])
    p2 = jnp.zeros((C, C), jnp.float32).at[:, 5:106].set(W_cls[:C])
    pp = jnp.zeros((2, C), jnp.float32)
    pp = pp.at[:, 0:4].set(W_reg[C:]).at[:, 4:5].set(W_obj[C:])
    pp = pp.at[:, 5:106].set(W_cls[C:])
    bh = jnp.zeros((C,), jnp.float32)
    bh = bh.at[0:4].set(b_reg).at[4:5].set(b_obj).at[5:106].set(b_cls)

    ah, dh = _dense3(s_mt[:C].T, d1, g1.reshape(1, C), be1.reshape(1, C),
                     s_mt[C:].T, d2, g2.reshape(1, C), be2.reshape(1, C),
                     pos2, p1, p2, pp, bh.reshape(1, C))
    s_ht = _segmax1(ah.T, src, dst)
    out = _final(s_ht.T, dh)

    reg = out[:, 0:4]
    obj = out[:, 4:5]
    cls = out[:, 5:106]
    return (cls, reg, obj)
